# baseline (speedup 1.0000x reference)
import jax
import jax.numpy as jnp
from jax import lax
from jax.experimental import pallas as pl
from jax.experimental.pallas import tpu as pltpu
from jax.experimental.pallas import tpu_sc as plsc

U = 512
B = 128
D = 8
C = 2
K = 8192
W = D * U
OUT = B * U
ST = (D - 1) * U
NK = (OUT + ST) // U
L = 16
NG = U // L
NROW = B * C * K // U

_NWPC = 15
_NW = C * _NWPC
_CPW = NK // _NWPC


def _sc_body(upd_hbm, st_hbm, out0_hbm, out1_hbm,
             idx_v, buf, sbuf, obuf, sem):
    wid = lax.axis_index("s") * 2 + lax.axis_index("c")

    @pl.when(wid < _NW)
    def _():
        c = wid // _NWPC
        k0 = (wid % _NWPC) * _CPW

        z = jnp.zeros((L,), jnp.float32)
        for j in range(2):
            for i in range(NG):
                sbuf[j, pl.ds(i * L, L)] = z

        lane = lax.iota(jnp.int32, L)
        jb = lane >> 3
        dd = lane & 7

        def do_pair(k, second):
            b = jnp.clip(k + jb - dd, 0, B - 1)
            idx_v[...] = (b * C + c) * (K // U) + D + dd
            gather = pltpu.make_async_copy(upd_hbm.at[idx_v], buf, sem)
            gather.start()

            nch = 2 if second else 1
            for j in range(nch):
                @pl.when(k + j < D - 1)
                def _(j=j):
                    pltpu.make_async_copy(
                        st_hbm.at[c, pl.ds((k + j) * U, U)], sbuf.at[j], sem
                    ).start()

            gather.wait()
            for j in range(nch):
                @pl.when(k + j < D - 1)
                def _(j=j):
                    pltpu.make_async_copy(
                        st_hbm.at[0, pl.ds(0, U)], sbuf.at[j], sem
                    ).wait()

            @pl.when((k < D - 1) | (k + 1 > B - 1))
            def _():
                for j in range(2):
                    for d in range(D):
                        @pl.when((k + j - d < 0) | (k + j - d > B - 1))
                        def _(j=j, d=d):
                            for i in range(NG):
                                buf[j * D + d, pl.ds(i * L, L)] = z

            for j in range(nch):
                for i in range(NG):
                    g = pl.ds(i * L, L)
                    s01 = buf[j * D + 0, g] + buf[j * D + 1, g]
                    s23 = buf[j * D + 2, g] + buf[j * D + 3, g]
                    s45 = buf[j * D + 4, g] + buf[j * D + 5, g]
                    s67 = buf[j * D + 6, g] + buf[j * D + 7, g]
                    s = (s01 + s23) + (s45 + s67)
                    obuf[pl.ds(j * U + i * L, L)] = (
                        s * jnp.float32(1.0 / D) + sbuf[j, g]
                    )
                @pl.when(k + j < D - 1)
                def _(j=j):
                    for i in range(NG):
                        sbuf[j, pl.ds(i * L, L)] = z

            sz = nch * U
            @pl.when(k < B)
            def _():
                pltpu.sync_copy(obuf.at[pl.ds(0, sz)],
                                out0_hbm.at[c, pl.ds(k * U, sz)])

            @pl.when(k >= B)
            def _():
                pltpu.sync_copy(obuf.at[pl.ds(0, sz)],
                                out1_hbm.at[c, pl.ds((k - B) * U, sz)])

        def pair_body(t, carry):
            do_pair(k0 + 2 * t, True)
            return carry

        lax.fori_loop(0, _CPW // 2, pair_body, 0)
        do_pair(k0 + _CPW - 1, False)


@jax.jit
def kernel(update, state):
    upd_rows = update.reshape(NROW, U)
    mesh = plsc.VectorSubcoreMesh(core_axis_name="c", subcore_axis_name="s")
    return pl.kernel(
        _sc_body,
        out_type=(
            jax.ShapeDtypeStruct((C, OUT), jnp.float32),
            jax.ShapeDtypeStruct((C, ST), jnp.float32),
        ),
        mesh=mesh,
        scratch_types=[
            pltpu.VMEM((L,), jnp.int32),
            pltpu.VMEM((2 * D, U), jnp.float32),
            pltpu.VMEM((2, U), jnp.float32),
            pltpu.VMEM((2 * U,), jnp.float32),
            pltpu.SemaphoreType.DMA,
        ],
    )(upd_rows, state)


# speedup vs baseline: 1.4144x; 1.4144x over previous
import jax
import jax.numpy as jnp
from jax import lax
from jax.experimental import pallas as pl
from jax.experimental.pallas import tpu as pltpu
from jax.experimental.pallas import tpu_sc as plsc

U = 512
B = 128
D = 8
C = 2
K = 8192
W = D * U
OUT = B * U
ST = (D - 1) * U
NK = (OUT + ST) // U
L = 16
NG = U // L

_NWPC = 15
_NW = C * _NWPC
_CPW = NK // _NWPC
_HALO = 16
_ZR = _HALO


def _sc_body(upd_hbm, st_hbm, out0_hbm, out1_hbm, buf, sbuf, obuf, sem):
    wid = lax.axis_index("s") * 2 + lax.axis_index("c")

    @pl.when(wid < _NW)
    def _():
        c = wid // _NWPC
        k0 = (wid % _NWPC) * _CPW
        bs = jnp.clip(k0 - (D - 1), 0, B - _HALO)

        fetch = pltpu.make_async_copy(
            upd_hbm.at[pl.ds(bs, _HALO), c, pl.ds(W, W)],
            buf.at[pl.ds(0, _HALO)],
            sem,
        )
        fetch.start()

        @pl.when(k0 == 0)
        def _():
            pltpu.make_async_copy(st_hbm.at[c], sbuf, sem).start()

        z = jnp.zeros((L,), jnp.float32)
        for i in range(W // L):
            buf[_ZR, pl.ds(i * L, L)] = z

        fetch.wait()

        @pl.when(k0 == 0)
        def _():
            pltpu.make_async_copy(st_hbm.at[0], sbuf, sem).wait()

        def chunk_body(j, carry):
            k = k0 + j
            rows = []
            for d in range(D):
                ok = (k - d >= 0) & (k - d <= B - 1)
                rows.append(jnp.where(ok, k - d - bs, _ZR))
            for i in range(NG):
                col = i * L
                s01 = (buf[rows[0], pl.ds(0 * U + col, L)]
                       + buf[rows[1], pl.ds(1 * U + col, L)])
                s23 = (buf[rows[2], pl.ds(2 * U + col, L)]
                       + buf[rows[3], pl.ds(3 * U + col, L)])
                s45 = (buf[rows[4], pl.ds(4 * U + col, L)]
                       + buf[rows[5], pl.ds(5 * U + col, L)])
                s67 = (buf[rows[6], pl.ds(6 * U + col, L)]
                       + buf[rows[7], pl.ds(7 * U + col, L)])
                s = (s01 + s23) + (s45 + s67)
                obuf[pl.ds(j * U + col, L)] = s * jnp.float32(1.0 / D)

            @pl.when(k < D - 1)
            def _():
                for i in range(NG):
                    g = pl.ds(j * U + i * L, L)
                    obuf[g] = obuf[g] + sbuf[pl.ds(k * U + i * L, L)]

            return carry

        lax.fori_loop(0, _CPW, chunk_body, 0)

        @pl.when(k0 + _CPW <= B)
        def _():
            pltpu.sync_copy(obuf, out0_hbm.at[c, pl.ds(k0 * U, _CPW * U)])

        @pl.when(k0 + _CPW > B)
        def _():
            head = (B - (_NWPC - 1) * _CPW) * U
            pltpu.sync_copy(obuf.at[pl.ds(0, head)],
                            out0_hbm.at[c, pl.ds((B * U - head), head)])
            pltpu.sync_copy(obuf.at[pl.ds(head, ST)], out1_hbm.at[c])


@jax.jit
def kernel(update, state):
    mesh = plsc.VectorSubcoreMesh(core_axis_name="c", subcore_axis_name="s")
    return pl.kernel(
        _sc_body,
        out_type=(
            jax.ShapeDtypeStruct((C, OUT), jnp.float32),
            jax.ShapeDtypeStruct((C, ST), jnp.float32),
        ),
        mesh=mesh,
        scratch_types=[
            pltpu.VMEM((_HALO + 1, W), jnp.float32),
            pltpu.VMEM((ST,), jnp.float32),
            pltpu.VMEM((_CPW * U,), jnp.float32),
            pltpu.SemaphoreType.DMA,
        ],
    )(update, state)
